# baseline (speedup 1.0000x reference)
import functools

import jax
import jax.numpy as jnp
from jax import lax
from jax.experimental import pallas as pl
from jax.experimental.pallas import tpu as pltpu
from jax.experimental.pallas import tpu_sc as plsc


def _sc_body(num_cores, rows_per_w, table_hbm, out_hbm, buf, sem):
    wid = lax.axis_index("s") * num_cores + lax.axis_index("c")
    base = wid * rows_per_w
    pltpu.sync_copy(table_hbm.at[pl.ds(base, rows_per_w)], buf)
    pltpu.sync_copy(buf, out_hbm.at[0, pl.ds(base, rows_per_w)])


@functools.cache
def _make_sc(num_rows, d_model, dtype):
    info = plsc.get_sparse_core_info()
    num_workers = info.num_cores * info.num_subcores
    rows_per_w = num_rows // num_workers
    mesh = plsc.VectorSubcoreMesh(core_axis_name="c", subcore_axis_name="s")
    return pl.kernel(
        functools.partial(_sc_body, info.num_cores, rows_per_w),
        mesh=mesh,
        out_type=jax.ShapeDtypeStruct((1, num_rows, d_model), dtype),
        scratch_types=[
            pltpu.VMEM((rows_per_w, d_model), dtype),
            pltpu.SemaphoreType.DMA,
        ],
    )


def _tc_body(nb, in_ref, out_ref):
    t = in_ref[...]
    for b in range(nb):
        out_ref[b] = t


@functools.cache
def _make_tc(nb, num_rows, d_model, dtype, block_rows=256):
    grid = (num_rows // block_rows,)
    return pl.pallas_call(
        functools.partial(_tc_body, nb),
        grid=grid,
        in_specs=[pl.BlockSpec((block_rows, d_model), lambda i: (i, 0))],
        out_specs=pl.BlockSpec((nb, block_rows, d_model), lambda i: (0, i, 0)),
        out_shape=jax.ShapeDtypeStruct((nb, num_rows, d_model), dtype),
    )


def _scs_body(batch, table_hbm, out_hbm, sem):
    cid = lax.axis_index("c")
    half = batch // 2
    for core in range(2):
        @pl.when(cid == core)
        def _(core=core):
            copies = [
                pltpu.async_copy(table_hbm, out_hbm.at[b], sem)
                for b in range(core * half, (core + 1) * half)
            ]
            for cp in copies:
                cp.wait()


@functools.cache
def _make_scs(batch, num_rows, d_model, dtype):
    mesh = plsc.ScalarSubcoreMesh(axis_name="c", num_cores=2)
    return pl.kernel(
        functools.partial(_scs_body, batch),
        mesh=mesh,
        out_type=jax.ShapeDtypeStruct((batch, num_rows, d_model), dtype),
        scratch_types=[pltpu.SemaphoreType.DMA],
    )


def kernel(x, table):
    batch, seq_len = x.shape
    num_rows, d_model = table.shape
    return _make_scs(batch, seq_len, d_model, table.dtype)(table[:seq_len])


# speedup vs baseline: 24.9077x; 24.9077x over previous
import functools

import jax
import jax.numpy as jnp
from jax import lax
from jax.experimental import pallas as pl
from jax.experimental.pallas import tpu as pltpu
from jax.experimental.pallas import tpu_sc as plsc


def _body(ns, rows_per_w, batch, table_hbm, out_hbm, buf, spmem, rsem, wsem, fsem, ssem):
    cid = lax.axis_index("c")
    sid = lax.axis_index("s")
    wid = cid * ns + sid
    base = wid * rows_per_w
    sc_rows = ns * rows_per_w
    sc_base = cid * sc_rows

    @pl.when(sid == 0)
    def _():
        pltpu.async_copy(table_hbm.at[pl.ds(sc_base, sc_rows)], spmem, fsem)

    pltpu.sync_copy(table_hbm.at[pl.ds(base, rows_per_w)], buf)
    writes = [
        pltpu.async_copy(buf, out_hbm.at[b, pl.ds(base, rows_per_w)], wsem)
        for b in range(batch - 1)
    ]

    @pl.when(sid == 0)
    def _():
        pltpu.make_async_copy(
            table_hbm.at[pl.ds(sc_base, sc_rows)], spmem, fsem
        ).wait()
        pltpu.async_copy(
            spmem, out_hbm.at[batch - 1, pl.ds(sc_base, sc_rows)], ssem
        ).wait()

    for cp in writes:
        cp.wait()


@functools.cache
def _make(batch, num_rows, d_model, dtype):
    info = plsc.get_sparse_core_info()
    nc, ns = info.num_cores, info.num_subcores
    rows_per_w = num_rows // (nc * ns)
    mesh = plsc.VectorSubcoreMesh(core_axis_name="c", subcore_axis_name="s")
    return pl.kernel(
        functools.partial(_body, ns, rows_per_w, batch),
        mesh=mesh,
        out_type=jax.ShapeDtypeStruct((batch, num_rows, d_model), dtype),
        scratch_types=[
            pltpu.VMEM((rows_per_w, d_model), dtype),
            pltpu.VMEM_SHARED((ns * rows_per_w, d_model), dtype),
            pltpu.SemaphoreType.DMA,
            pltpu.SemaphoreType.DMA,
            pltpu.SemaphoreType.DMA,
            pltpu.SemaphoreType.DMA,
        ],
    )


def kernel(x, table):
    batch, seq_len = x.shape
    num_rows, d_model = table.shape
    return _make(batch, seq_len, d_model, table.dtype)(table[:seq_len])


# speedup vs baseline: 26.7550x; 1.0742x over previous
import functools

import jax
import jax.numpy as jnp
from jax import lax
from jax.experimental import pallas as pl
from jax.experimental.pallas import tpu as pltpu
from jax.experimental.pallas import tpu_sc as plsc


_N_CHUNKS = 4


def _broadcast_body(num_cores, rows_per_w, batch, table_hbm, out_hbm, *rest):
    bufs, (rsem, wsem) = rest[:_N_CHUNKS], rest[_N_CHUNKS:]
    chunk = rows_per_w // _N_CHUNKS
    wid = lax.axis_index("s") * num_cores + lax.axis_index("c")
    base = wid * rows_per_w
    reads = [
        pltpu.async_copy(table_hbm.at[pl.ds(base + c * chunk, chunk)], bufs[c], rsem)
        for c in range(_N_CHUNKS)
    ]
    writes = []
    for c in range(_N_CHUNKS):
        reads[c].wait()
        writes += [
            pltpu.async_copy(
                bufs[c], out_hbm.at[b, pl.ds(base + c * chunk, chunk)], wsem
            )
            for b in range(batch)
        ]
    for cp in writes:
        cp.wait()


@functools.cache
def _make_broadcast(batch, num_rows, d_model, dtype):
    info = plsc.get_sparse_core_info()
    num_workers = info.num_cores * info.num_subcores
    assert num_rows % (num_workers * _N_CHUNKS) == 0
    rows_per_w = num_rows // num_workers
    mesh = plsc.VectorSubcoreMesh(core_axis_name="c", subcore_axis_name="s")
    return pl.kernel(
        functools.partial(_broadcast_body, info.num_cores, rows_per_w, batch),
        mesh=mesh,
        out_type=jax.ShapeDtypeStruct((batch, num_rows, d_model), dtype),
        scratch_types=[
            *[
                pltpu.VMEM((rows_per_w // _N_CHUNKS, d_model), dtype)
                for _ in range(_N_CHUNKS)
            ],
            pltpu.SemaphoreType.DMA,
            pltpu.SemaphoreType.DMA,
        ],
    )


def kernel(x, table):
    batch, seq_len = x.shape
    num_rows, d_model = table.shape
    fn = _make_broadcast(batch, seq_len, d_model, table.dtype)
    return fn(table[:seq_len])
